# baseline (speedup 1.0000x reference)
import jax
import jax.numpy as jnp
from jax import lax
from jax.experimental import pallas as pl
from jax.experimental.pallas import tpu as pltpu
from jax.experimental.pallas import tpu_sc as plsc

L = 16
NCORES = 2
NSUB = 16
NQ = 4
NPAIR = 8
NEG_INF_PACKED = -8388736


def _double_buffered(nchunks, start, wait, work):
    start(0, 0)
    if nchunks > 1:
        start(1, 1)

    def pair(k2, _):
        for b in range(2):
            k = k2 * 2 + b
            wait(k, b)
            work(k, b)

            @pl.when(k + 2 < nchunks)
            def _prefetch():
                start(k + 2, b)
        return 0

    if nchunks // 2 > 0:
        lax.fori_loop(0, nchunks // 2, pair, 0)
    if nchunks % 2:
        k = nchunks - 1
        wait(k, k % 2)
        work(k, k % 2)


def _lo_f32(w):
    return plsc.bitcast(w << 16, jnp.float32)


def _hi_f32(w):
    return plsc.bitcast(w & jnp.int32(-65536), jnp.float32)


def _sc_segment_max(nodes, adj, w_theta, n_nodes, n_edges):
    N = n_nodes
    EQ = n_edges // NQ
    ECH = 3200 if EQ % 3200 == 0 else EQ
    NCH = 2000 if N % 2000 == 0 else N
    n_echunks = EQ // ECH
    n_nchunks = N // NCH

    def body(nodes_hbm, adj_hbm, w_hbm, m_hbm,
             w_v, nbuf0, nbuf1, pp_v, out_v,
             esrc0, esrc1, edst0, edst1, sem_n, sem_e):
        nbufs = (nbuf0, nbuf1)
        esrcs = (esrc0, esrc1)
        edsts = (edst0, edst1)
        c = lax.axis_index("c")
        s = lax.axis_index("s")
        j = s % NPAIR
        q = c * 2 + s // NPAIR
        iota = lax.iota(jnp.int32, L)

        pltpu.sync_copy(w_hbm, w_v)
        ja = jnp.full((L,), j, jnp.int32)
        wa = [plsc.load_gather(w_v, [ja + r * 16]) for r in range(3)]
        wb = [plsc.load_gather(w_v, [ja + (8 + r * 16)]) for r in range(3)]

        ninf = jnp.full((L,), NEG_INF_PACKED, jnp.int32)

        def init(i, _):
            out_v[pl.ds(i * L, L)] = ninf
            return 0

        lax.fori_loop(0, N // L, init, 0)

        def nstart(k, b):
            pltpu.async_copy(nodes_hbm.at[pl.ds(k * NCH * 3, NCH * 3)],
                             nbufs[b], sem_n.at[b])

        def nwait(k, b):
            pltpu.make_async_copy(nodes_hbm.at[pl.ds(k * NCH * 3, NCH * 3)],
                                  nbufs[b], sem_n.at[b]).wait()

        def nwork(k, b):
            def pg(g, _):
                fidx = (g * L + iota) * 3
                x0 = plsc.load_gather(nbufs[b], [fidx])
                x1 = plsc.load_gather(nbufs[b], [fidx + 1])
                x2 = plsc.load_gather(nbufs[b], [fidx + 2])
                pa = x0 * wa[0] + x1 * wa[1] + x2 * wa[2]
                pb = x0 * wb[0] + x1 * wb[1] + x2 * wb[2]
                packed = plsc.pack(pa, pb, format=plsc.PackFormat.INTERLEAVED)
                pp_v[pl.ds(k * NCH + g * L, L)] = plsc.bitcast(packed,
                                                               jnp.int32)
                return 0

            lax.fori_loop(0, NCH // L, pg, 0)

        _double_buffered(n_nchunks, nstart, nwait, nwork)

        ebase = q * EQ

        def estart(k, b):
            off = ebase + k * ECH
            pltpu.async_copy(adj_hbm.at[0, pl.ds(off, ECH)],
                             esrcs[b], sem_e.at[b])
            pltpu.async_copy(adj_hbm.at[1, pl.ds(off, ECH)],
                             edsts[b], sem_e.at[b])

        def ewait(k, b):
            off = ebase + k * ECH
            pltpu.make_async_copy(adj_hbm.at[0, pl.ds(off, ECH)],
                                  esrcs[b], sem_e.at[b]).wait()
            pltpu.make_async_copy(adj_hbm.at[1, pl.ds(off, ECH)],
                                  edsts[b], sem_e.at[b]).wait()

        def pmax(cur_w, val_bf):
            cur_bf = plsc.bitcast(cur_w, jnp.bfloat16)
            return plsc.bitcast(jnp.maximum(cur_bf, val_bf), jnp.int32)

        U = 20
        assert (ECH // L) % U == 0

        def ework(_k, b):
            def grp(gb, _):
                svs, vals, fails = [], [], []
                for u in range(U):
                    g = gb * U + u
                    sv = esrcs[b][pl.ds(g * L, L)]
                    dv = edsts[b][pl.ds(g * L, L)]
                    val_bf = plsc.bitcast(plsc.load_gather(pp_v, [dv]),
                                          jnp.bfloat16)
                    cur_w = plsc.load_gather(out_v, [sv])
                    plsc.store_scatter(out_v, [sv], pmax(cur_w, val_bf))
                    svs.append(sv)
                    vals.append(val_bf)
                fail_or = None
                for u in range(U):
                    chk_w = plsc.load_gather(out_v, [svs[u]])
                    f = pmax(chk_w, vals[u]) != chk_w
                    fails.append(f)
                    fail_or = f if fail_or is None else (fail_or | f)

                @pl.when(jnp.any(fail_or))
                def _slow():
                    def cond(carry):
                        return carry[0]

                    def rbody(carry):
                        _, ms = carry
                        for u in range(U):
                            cur_w = plsc.load_gather(out_v, [svs[u]])
                            plsc.store_scatter(out_v, [svs[u]],
                                               pmax(cur_w, vals[u]),
                                               mask=ms[u])
                        nms, anyf = [], None
                        for u in range(U):
                            chk_w = plsc.load_gather(out_v, [svs[u]])
                            f = ms[u] & (pmax(chk_w, vals[u]) != chk_w)
                            nms.append(f)
                            anyf = f if anyf is None else (anyf | f)
                        return (jnp.any(anyf), tuple(nms))

                    lax.while_loop(cond, rbody,
                                   (jnp.any(fail_or), tuple(fails)))

                return 0

            lax.fori_loop(0, ECH // (L * U), grp, 0)

        _double_buffered(n_echunks, estart, ewait, ework)

        def fchunk(kk, _):
            def fg(g, _g):
                sl = pl.ds(kk * NCH + g * L, L)
                ow = out_v[sl]
                pw = pp_v[sl]
                lsl = pl.ds(g * L, L)
                nbuf0[lsl] = _lo_f32(ow) - _lo_f32(pw)
                nbuf1[lsl] = _hi_f32(ow) - _hi_f32(pw)
                return 0

            lax.fori_loop(0, NCH // L, fg, 0)
            row = (q * NSUB + j) * N
            pltpu.sync_copy(nbuf0.at[pl.ds(0, NCH)],
                            m_hbm.at[pl.ds(row + kk * NCH, NCH)])
            pltpu.sync_copy(nbuf1.at[pl.ds(0, NCH)],
                            m_hbm.at[pl.ds(row + 8 * N + kk * NCH, NCH)])
            return 0

        lax.fori_loop(0, n_nchunks, fchunk, 0)

    kern = pl.kernel(
        body,
        out_type=jax.ShapeDtypeStruct((NQ * NSUB * N,), jnp.float32),
        mesh=plsc.VectorSubcoreMesh(core_axis_name="c", subcore_axis_name="s",
                                    num_cores=NCORES, num_subcores=NSUB),
        scratch_types=[
            pltpu.VMEM((3 * L,), jnp.float32),
            pltpu.VMEM((NCH * 3,), jnp.float32),
            pltpu.VMEM((NCH * 3,), jnp.float32),
            pltpu.VMEM((N,), jnp.int32),
            pltpu.VMEM((N,), jnp.int32),
            pltpu.VMEM((ECH,), jnp.int32),
            pltpu.VMEM((ECH,), jnp.int32),
            pltpu.VMEM((ECH,), jnp.int32),
            pltpu.VMEM((ECH,), jnp.int32),
            pltpu.SemaphoreType.DMA((2,)),
            pltpu.SemaphoreType.DMA((2,)),
        ],
        compiler_params=pltpu.CompilerParams(needs_layout_passes=False),
    )
    out = kern(nodes.reshape(-1), adj, w_theta.reshape(-1))
    return out.reshape(NQ, NSUB, N)


def _tc_combine(prev, m, w_phi, n):
    d = w_phi.shape[0]

    def body(prev_ref, m_ref, w_ref, o_ref):
        mx = jnp.maximum(jnp.maximum(m_ref[0], m_ref[1]),
                         jnp.maximum(m_ref[2], m_ref[3]))
        acc = jnp.sum(mx * w_ref[...], axis=0, keepdims=True)
        o_ref[...] = prev_ref[...] + acc * (1.0 / d)

    out = pl.pallas_call(
        body,
        out_shape=jax.ShapeDtypeStruct((1, n), jnp.float32),
    )(prev.reshape(1, n), m, w_phi.reshape(d, 1))
    return out.reshape(n)


def kernel(previous_inclusion_score, nodes, adjacency_indices, W_phi, W_theta):
    n = nodes.shape[0]
    e = adjacency_indices.shape[1]
    m = _sc_segment_max(nodes, adjacency_indices, W_theta, n, e)
    return _tc_combine(previous_inclusion_score, m, W_phi, n)


# speedup vs baseline: 1.0041x; 1.0041x over previous
import jax
import jax.numpy as jnp
from jax import lax
from jax.experimental import pallas as pl
from jax.experimental.pallas import tpu as pltpu
from jax.experimental.pallas import tpu_sc as plsc

L = 16
NCORES = 2
NSUB = 16
NQ = 4
NPAIR = 8
NEG_INF_PACKED = -8388736


def _double_buffered(nchunks, start, wait, work):
    start(0, 0)
    if nchunks > 1:
        start(1, 1)

    def pair(k2, _):
        for b in range(2):
            k = k2 * 2 + b
            wait(k, b)
            work(k, b)

            @pl.when(k + 2 < nchunks)
            def _prefetch():
                start(k + 2, b)
        return 0

    if nchunks // 2 > 0:
        lax.fori_loop(0, nchunks // 2, pair, 0)
    if nchunks % 2:
        k = nchunks - 1
        wait(k, k % 2)
        work(k, k % 2)


def _lo_f32(w):
    return plsc.bitcast(w << 16, jnp.float32)


def _hi_f32(w):
    return plsc.bitcast(w & jnp.int32(-65536), jnp.float32)


def _sc_segment_max(nodes, adj, w_theta, n_nodes, n_edges):
    N = n_nodes
    EQ = n_edges // NQ
    ECH = 3200 if EQ % 3200 == 0 else EQ
    NCH = 2000 if N % 2000 == 0 else N
    n_echunks = EQ // ECH
    n_nchunks = N // NCH

    def body(nodes_hbm, adj_hbm, w_hbm, m_hbm,
             w_v, nbuf0, nbuf1, pp_v, out_v,
             esrc0, esrc1, edst0, edst1, sem_n, sem_e):
        nbufs = (nbuf0, nbuf1)
        esrcs = (esrc0, esrc1)
        edsts = (edst0, edst1)
        c = lax.axis_index("c")
        s = lax.axis_index("s")
        j = s % NPAIR
        q = c * 2 + s // NPAIR
        iota = lax.iota(jnp.int32, L)

        pltpu.sync_copy(w_hbm, w_v)
        ja = jnp.full((L,), j, jnp.int32)
        wa = [plsc.load_gather(w_v, [ja + r * 16]) for r in range(3)]
        wb = [plsc.load_gather(w_v, [ja + (8 + r * 16)]) for r in range(3)]

        ninf = jnp.full((L,), NEG_INF_PACKED, jnp.int32)

        def init(i, _):
            out_v[pl.ds(i * L, L)] = ninf
            return 0

        lax.fori_loop(0, N // L, init, 0)

        def nstart(k, b):
            pltpu.async_copy(nodes_hbm.at[pl.ds(k * NCH * 3, NCH * 3)],
                             nbufs[b], sem_n.at[b])

        def nwait(k, b):
            pltpu.make_async_copy(nodes_hbm.at[pl.ds(k * NCH * 3, NCH * 3)],
                                  nbufs[b], sem_n.at[b]).wait()

        def nwork(k, b):
            def pg(g, _):
                fidx = (g * L + iota) * 3
                x0 = plsc.load_gather(nbufs[b], [fidx])
                x1 = plsc.load_gather(nbufs[b], [fidx + 1])
                x2 = plsc.load_gather(nbufs[b], [fidx + 2])
                pa = x0 * wa[0] + x1 * wa[1] + x2 * wa[2]
                pb = x0 * wb[0] + x1 * wb[1] + x2 * wb[2]
                packed = plsc.pack(pa, pb, format=plsc.PackFormat.INTERLEAVED)
                pp_v[pl.ds(k * NCH + g * L, L)] = plsc.bitcast(packed,
                                                               jnp.int32)
                return 0

            lax.fori_loop(0, NCH // L, pg, 0)

        _double_buffered(n_nchunks, nstart, nwait, nwork)

        ebase = q * EQ

        def estart(k, b):
            off = ebase + k * ECH
            pltpu.async_copy(adj_hbm.at[0, pl.ds(off, ECH)],
                             esrcs[b], sem_e.at[b])
            pltpu.async_copy(adj_hbm.at[1, pl.ds(off, ECH)],
                             edsts[b], sem_e.at[b])

        def ewait(k, b):
            off = ebase + k * ECH
            pltpu.make_async_copy(adj_hbm.at[0, pl.ds(off, ECH)],
                                  esrcs[b], sem_e.at[b]).wait()
            pltpu.make_async_copy(adj_hbm.at[1, pl.ds(off, ECH)],
                                  edsts[b], sem_e.at[b]).wait()

        def pmax(cur_w, val_bf):
            cur_bf = plsc.bitcast(cur_w, jnp.bfloat16)
            return plsc.bitcast(jnp.maximum(cur_bf, val_bf), jnp.int32)

        U = 25
        assert (ECH // L) % U == 0

        def ework(_k, b):
            def grp(gb, _):
                svs, vals, fails = [], [], []
                for u in range(U):
                    g = gb * U + u
                    sv = esrcs[b][pl.ds(g * L, L)]
                    dv = edsts[b][pl.ds(g * L, L)]
                    val_bf = plsc.bitcast(plsc.load_gather(pp_v, [dv]),
                                          jnp.bfloat16)
                    cur_w = plsc.load_gather(out_v, [sv])
                    plsc.store_scatter(out_v, [sv], pmax(cur_w, val_bf))
                    svs.append(sv)
                    vals.append(val_bf)
                fail_or = None
                for u in range(U):
                    chk_w = plsc.load_gather(out_v, [svs[u]])
                    f = pmax(chk_w, vals[u]) != chk_w
                    fails.append(f)
                    fail_or = f if fail_or is None else (fail_or | f)

                @pl.when(jnp.any(fail_or))
                def _slow():
                    def cond(carry):
                        return carry[0]

                    def rbody(carry):
                        _, ms = carry
                        for u in range(U):
                            cur_w = plsc.load_gather(out_v, [svs[u]])
                            plsc.store_scatter(out_v, [svs[u]],
                                               pmax(cur_w, vals[u]),
                                               mask=ms[u])
                        nms, anyf = [], None
                        for u in range(U):
                            chk_w = plsc.load_gather(out_v, [svs[u]])
                            f = ms[u] & (pmax(chk_w, vals[u]) != chk_w)
                            nms.append(f)
                            anyf = f if anyf is None else (anyf | f)
                        return (jnp.any(anyf), tuple(nms))

                    lax.while_loop(cond, rbody,
                                   (jnp.any(fail_or), tuple(fails)))

                return 0

            lax.fori_loop(0, ECH // (L * U), grp, 0)

        _double_buffered(n_echunks, estart, ewait, ework)

        def fchunk(kk, _):
            def fg(g, _g):
                sl = pl.ds(kk * NCH + g * L, L)
                ow = out_v[sl]
                pw = pp_v[sl]
                lsl = pl.ds(g * L, L)
                nbuf0[lsl] = _lo_f32(ow) - _lo_f32(pw)
                nbuf1[lsl] = _hi_f32(ow) - _hi_f32(pw)
                return 0

            lax.fori_loop(0, NCH // L, fg, 0)
            row = (q * NSUB + j) * N
            pltpu.sync_copy(nbuf0.at[pl.ds(0, NCH)],
                            m_hbm.at[pl.ds(row + kk * NCH, NCH)])
            pltpu.sync_copy(nbuf1.at[pl.ds(0, NCH)],
                            m_hbm.at[pl.ds(row + 8 * N + kk * NCH, NCH)])
            return 0

        lax.fori_loop(0, n_nchunks, fchunk, 0)

    kern = pl.kernel(
        body,
        out_type=jax.ShapeDtypeStruct((NQ * NSUB * N,), jnp.float32),
        mesh=plsc.VectorSubcoreMesh(core_axis_name="c", subcore_axis_name="s",
                                    num_cores=NCORES, num_subcores=NSUB),
        scratch_types=[
            pltpu.VMEM((3 * L,), jnp.float32),
            pltpu.VMEM((NCH * 3,), jnp.float32),
            pltpu.VMEM((NCH * 3,), jnp.float32),
            pltpu.VMEM((N,), jnp.int32),
            pltpu.VMEM((N,), jnp.int32),
            pltpu.VMEM((ECH,), jnp.int32),
            pltpu.VMEM((ECH,), jnp.int32),
            pltpu.VMEM((ECH,), jnp.int32),
            pltpu.VMEM((ECH,), jnp.int32),
            pltpu.SemaphoreType.DMA((2,)),
            pltpu.SemaphoreType.DMA((2,)),
        ],
        compiler_params=pltpu.CompilerParams(needs_layout_passes=False),
    )
    out = kern(nodes.reshape(-1), adj, w_theta.reshape(-1))
    return out.reshape(NQ, NSUB, N)


def _tc_combine(prev, m, w_phi, n):
    d = w_phi.shape[0]

    def body(prev_ref, m_ref, w_ref, o_ref):
        mx = jnp.maximum(jnp.maximum(m_ref[0], m_ref[1]),
                         jnp.maximum(m_ref[2], m_ref[3]))
        acc = jnp.sum(mx * w_ref[...], axis=0, keepdims=True)
        o_ref[...] = prev_ref[...] + acc * (1.0 / d)

    out = pl.pallas_call(
        body,
        out_shape=jax.ShapeDtypeStruct((1, n), jnp.float32),
    )(prev.reshape(1, n), m, w_phi.reshape(d, 1))
    return out.reshape(n)


def kernel(previous_inclusion_score, nodes, adjacency_indices, W_phi, W_theta):
    n = nodes.shape[0]
    e = adjacency_indices.shape[1]
    m = _sc_segment_max(nodes, adjacency_indices, W_theta, n, e)
    return _tc_combine(previous_inclusion_score, m, W_phi, n)


# speedup vs baseline: 1.0175x; 1.0134x over previous
import jax
import jax.numpy as jnp
from jax import lax
from jax.experimental import pallas as pl
from jax.experimental.pallas import tpu as pltpu
from jax.experimental.pallas import tpu_sc as plsc

L = 16
NCORES = 2
NSUB = 16
NQ = 4
NPAIR = 8
NEG_INF_PACKED = -8388736


def _double_buffered(nchunks, start, wait, work):
    start(0, 0)
    if nchunks > 1:
        start(1, 1)

    def pair(k2, _):
        for b in range(2):
            k = k2 * 2 + b
            wait(k, b)
            work(k, b)

            @pl.when(k + 2 < nchunks)
            def _prefetch():
                start(k + 2, b)
        return 0

    if nchunks // 2 > 0:
        lax.fori_loop(0, nchunks // 2, pair, 0)
    if nchunks % 2:
        k = nchunks - 1
        wait(k, k % 2)
        work(k, k % 2)


def _lo_f32(w):
    return plsc.bitcast(w << 16, jnp.float32)


def _hi_f32(w):
    return plsc.bitcast(w & jnp.int32(-65536), jnp.float32)


def _sc_segment_max(nodes, adj, w_theta, n_nodes, n_edges):
    N = n_nodes
    EQ = n_edges // NQ
    ECH = 3200 if EQ % 3200 == 0 else EQ
    NCH = 2000 if N % 2000 == 0 else N
    n_echunks = EQ // ECH
    n_nchunks = N // NCH

    def body(nodes_hbm, adj_hbm, w_hbm, m_hbm,
             w_v, nbuf0, nbuf1, pp_v, out_v,
             esrc0, esrc1, edst0, edst1, sem_n, sem_e):
        nbufs = (nbuf0, nbuf1)
        esrcs = (esrc0, esrc1)
        edsts = (edst0, edst1)
        c = lax.axis_index("c")
        s = lax.axis_index("s")
        j = s % NPAIR
        q = c * 2 + s // NPAIR
        iota = lax.iota(jnp.int32, L)

        pltpu.sync_copy(w_hbm, w_v)
        ja = jnp.full((L,), j, jnp.int32)
        wa = [plsc.load_gather(w_v, [ja + r * 16]) for r in range(3)]
        wb = [plsc.load_gather(w_v, [ja + (8 + r * 16)]) for r in range(3)]

        ninf = jnp.full((L,), NEG_INF_PACKED, jnp.int32)

        def init(i, _):
            out_v[pl.ds(i * L, L)] = ninf
            return 0

        lax.fori_loop(0, N // L, init, 0)

        def nstart(k, b):
            pltpu.async_copy(nodes_hbm.at[pl.ds(k * NCH * 3, NCH * 3)],
                             nbufs[b], sem_n.at[b])

        def nwait(k, b):
            pltpu.make_async_copy(nodes_hbm.at[pl.ds(k * NCH * 3, NCH * 3)],
                                  nbufs[b], sem_n.at[b]).wait()

        def nwork(k, b):
            def pg(g, _):
                fidx = (g * L + iota) * 3
                x0 = plsc.load_gather(nbufs[b], [fidx])
                x1 = plsc.load_gather(nbufs[b], [fidx + 1])
                x2 = plsc.load_gather(nbufs[b], [fidx + 2])
                pa = x0 * wa[0] + x1 * wa[1] + x2 * wa[2]
                pb = x0 * wb[0] + x1 * wb[1] + x2 * wb[2]
                packed = plsc.pack(pa, pb, format=plsc.PackFormat.INTERLEAVED)
                pp_v[pl.ds(k * NCH + g * L, L)] = plsc.bitcast(packed,
                                                               jnp.int32)
                return 0

            lax.fori_loop(0, NCH // L, pg, 0)

        _double_buffered(n_nchunks, nstart, nwait, nwork)

        ebase = q * EQ

        def estart(k, b):
            off = ebase + k * ECH
            pltpu.async_copy(adj_hbm.at[0, pl.ds(off, ECH)],
                             esrcs[b], sem_e.at[b])
            pltpu.async_copy(adj_hbm.at[1, pl.ds(off, ECH)],
                             edsts[b], sem_e.at[b])

        def ewait(k, b):
            off = ebase + k * ECH
            pltpu.make_async_copy(adj_hbm.at[0, pl.ds(off, ECH)],
                                  esrcs[b], sem_e.at[b]).wait()
            pltpu.make_async_copy(adj_hbm.at[1, pl.ds(off, ECH)],
                                  edsts[b], sem_e.at[b]).wait()

        def pmax(cur_w, val_bf):
            cur_bf = plsc.bitcast(cur_w, jnp.bfloat16)
            return plsc.bitcast(jnp.maximum(cur_bf, val_bf), jnp.int32)

        U = 25
        assert (ECH // L) % U == 0

        def ework(_k, b):
            def grp(gb, _):
                svs, vals, fails = [], [], []
                for u in range(U):
                    g = gb * U + u
                    sv = esrcs[b][pl.ds(g * L, L)]
                    dv = edsts[b][pl.ds(g * L, L)]
                    val_bf = plsc.bitcast(plsc.load_gather(pp_v, [dv]),
                                          jnp.bfloat16)
                    cur_w = plsc.load_gather(out_v, [sv])
                    plsc.store_scatter(out_v, [sv], pmax(cur_w, val_bf))
                    svs.append(sv)
                    vals.append(val_bf)
                fail_or = None
                for u in range(U):
                    chk_w = plsc.load_gather(out_v, [svs[u]])
                    f = pmax(chk_w, vals[u]) != chk_w
                    fails.append(f)
                    fail_or = f if fail_or is None else (fail_or | f)

                @pl.when(jnp.any(fail_or))
                def _slow():
                    def cond(carry):
                        return carry[0]

                    def rbody(carry):
                        _, ms = carry
                        for u in range(U):
                            cur_w = plsc.load_gather(out_v, [svs[u]])
                            plsc.store_scatter(out_v, [svs[u]],
                                               pmax(cur_w, vals[u]),
                                               mask=ms[u])
                        nms, anyf = [], None
                        for u in range(U):
                            chk_w = plsc.load_gather(out_v, [svs[u]])
                            f = ms[u] & (pmax(chk_w, vals[u]) != chk_w)
                            nms.append(f)
                            anyf = f if anyf is None else (anyf | f)
                        return (jnp.any(anyf), tuple(nms))

                    lax.while_loop(cond, rbody,
                                   (jnp.any(fail_or), tuple(fails)))

                return 0

            lax.fori_loop(0, ECH // (L * U), grp, 0)

        _double_buffered(n_echunks, estart, ewait, ework)

        row = (q * NSUB + j) * N

        def fstart(kk, b):
            pltpu.async_copy(nbufs[b].at[pl.ds(0, NCH)],
                             m_hbm.at[pl.ds(row + kk * NCH, NCH)],
                             sem_n.at[b])
            pltpu.async_copy(nbufs[b].at[pl.ds(NCH, NCH)],
                             m_hbm.at[pl.ds(row + 8 * N + kk * NCH, NCH)],
                             sem_n.at[b])

        def fwait(kk, b):
            pltpu.make_async_copy(nbufs[b].at[pl.ds(0, NCH)],
                                  m_hbm.at[pl.ds(row + kk * NCH, NCH)],
                                  sem_n.at[b]).wait()
            pltpu.make_async_copy(
                nbufs[b].at[pl.ds(NCH, NCH)],
                m_hbm.at[pl.ds(row + 8 * N + kk * NCH, NCH)],
                sem_n.at[b]).wait()

        def ffill(kk, b):
            def fg(g, _g):
                sl = pl.ds(kk * NCH + g * L, L)
                ow = out_v[sl]
                pw = pp_v[sl]
                nbufs[b][pl.ds(g * L, L)] = _lo_f32(ow) - _lo_f32(pw)
                nbufs[b][pl.ds(NCH + g * L, L)] = _hi_f32(ow) - _hi_f32(pw)
                return 0

            lax.fori_loop(0, NCH // L, fg, 0)

        def fpair(k2, _):
            for b in range(2):
                kk = k2 * 2 + b

                @pl.when(kk >= 2)
                def _drain():
                    fwait(kk - 2, b)

                ffill(kk, b)
                fstart(kk, b)
            return 0

        lax.fori_loop(0, n_nchunks // 2, fpair, 0)
        for b in range(n_nchunks % 2):
            kk = n_nchunks - 1
            if n_nchunks >= 3:
                fwait(kk - 2, kk % 2)
            ffill(kk, kk % 2)
            fstart(kk, kk % 2)
        if n_nchunks >= 2:
            fwait(n_nchunks - 2, (n_nchunks - 2) % 2)
        fwait(n_nchunks - 1, (n_nchunks - 1) % 2)

    kern = pl.kernel(
        body,
        out_type=jax.ShapeDtypeStruct((NQ * NSUB * N,), jnp.float32),
        mesh=plsc.VectorSubcoreMesh(core_axis_name="c", subcore_axis_name="s",
                                    num_cores=NCORES, num_subcores=NSUB),
        scratch_types=[
            pltpu.VMEM((3 * L,), jnp.float32),
            pltpu.VMEM((NCH * 3,), jnp.float32),
            pltpu.VMEM((NCH * 3,), jnp.float32),
            pltpu.VMEM((N,), jnp.int32),
            pltpu.VMEM((N,), jnp.int32),
            pltpu.VMEM((ECH,), jnp.int32),
            pltpu.VMEM((ECH,), jnp.int32),
            pltpu.VMEM((ECH,), jnp.int32),
            pltpu.VMEM((ECH,), jnp.int32),
            pltpu.SemaphoreType.DMA((2,)),
            pltpu.SemaphoreType.DMA((2,)),
        ],
        compiler_params=pltpu.CompilerParams(needs_layout_passes=False),
    )
    out = kern(nodes.reshape(-1), adj, w_theta.reshape(-1))
    return out.reshape(NQ, NSUB, N)


def _tc_combine(prev, m, w_phi, n):
    d = w_phi.shape[0]

    def body(prev_ref, m_ref, w_ref, o_ref):
        mx = jnp.maximum(jnp.maximum(m_ref[0], m_ref[1]),
                         jnp.maximum(m_ref[2], m_ref[3]))
        acc = jnp.sum(mx * w_ref[...], axis=0, keepdims=True)
        o_ref[...] = prev_ref[...] + acc * (1.0 / d)

    out = pl.pallas_call(
        body,
        out_shape=jax.ShapeDtypeStruct((1, n), jnp.float32),
    )(prev.reshape(1, n), m, w_phi.reshape(d, 1))
    return out.reshape(n)


def kernel(previous_inclusion_score, nodes, adjacency_indices, W_phi, W_theta):
    n = nodes.shape[0]
    e = adjacency_indices.shape[1]
    m = _sc_segment_max(nodes, adjacency_indices, W_theta, n, e)
    return _tc_combine(previous_inclusion_score, m, W_phi, n)
